# baseline (speedup 1.0000x reference)
import functools

import jax
import jax.numpy as jnp
from jax import lax
from jax.experimental import pallas as pl
from jax.experimental.pallas import tpu as pltpu
from jax.experimental.pallas import tpu_sc as plsc

N_TOKENS = 8192
D_MODEL = 768
HIDDEN = 256
NUM_EXPERTS = 16
TOP_K = 2

BLK = 2048

_NC = 2
_NS = 16
_NW = _NC * _NS
_RPW = N_TOKENS // _NW
_L = 16


def _mlp_body(x_ref, w1_ref, b1_ref, w2_ref, b2_ref, w3_ref, b3_ref,
              logits_ref):
    h = jnp.dot(x_ref[...], w1_ref[...], preferred_element_type=jnp.float32)
    h = jnp.maximum(h + b1_ref[...], 0.0)
    h = jnp.dot(h, w2_ref[...], preferred_element_type=jnp.float32)
    h = jnp.maximum(h + b2_ref[...], 0.0)
    lt = lax.dot_general(w3_ref[...], h, (((1,), (1,)), ((), ())),
                         preferred_element_type=jnp.float32)
    logits_ref[...] = lt + jnp.transpose(b3_ref[...])


def _mlp_logits_t(x, W1, b1, W2, b2, W3, b3):
    n = x.shape[0]
    full = lambda i: (0, 0)
    return pl.pallas_call(
        _mlp_body,
        grid=(n // BLK,),
        in_specs=[
            pl.BlockSpec((BLK, D_MODEL), lambda i: (i, 0)),
            pl.BlockSpec((D_MODEL, HIDDEN), full),
            pl.BlockSpec((1, HIDDEN), full),
            pl.BlockSpec((HIDDEN, HIDDEN // 2), full),
            pl.BlockSpec((1, HIDDEN // 2), full),
            pl.BlockSpec((NUM_EXPERTS, HIDDEN // 2), full),
            pl.BlockSpec((1, NUM_EXPERTS), full),
        ],
        out_specs=pl.BlockSpec((NUM_EXPERTS, BLK), lambda i: (0, i)),
        out_shape=jax.ShapeDtypeStruct((NUM_EXPERTS, n), jnp.float32),
    )(x, W1, b1.reshape(1, -1), W2, b2.reshape(1, -1), W3.T,
      b3.reshape(1, -1))


@functools.partial(
    pl.kernel,
    out_type=[
        jax.ShapeDtypeStruct((NUM_EXPERTS, N_TOKENS), jnp.float32),
        jax.ShapeDtypeStruct((TOP_K, N_TOKENS), jnp.int32),
    ],
    mesh=plsc.VectorSubcoreMesh(core_axis_name="c", subcore_axis_name="s"),
    compiler_params=pltpu.CompilerParams(needs_layout_passes=False),
    scratch_types=[
        pltpu.VMEM((NUM_EXPERTS, _RPW), jnp.float32),
        pltpu.VMEM((NUM_EXPERTS, _RPW), jnp.float32),
        pltpu.VMEM((TOP_K, _RPW), jnp.int32),
    ],
)
def _route_sc(lgt_hbm, gates_hbm, idx_hbm, lgt_v, gates_v, idx_v):
    wid = lax.axis_index("s") * _NC + lax.axis_index("c")
    base = wid * _RPW
    pltpu.sync_copy(lgt_hbm.at[:, pl.ds(base, _RPW)], lgt_v)

    def chunk(c, _):
        off = c * _L
        m1_0 = lgt_v[0, pl.ds(off, _L)]
        m2_0 = jnp.full((_L,), -jnp.inf, jnp.float32)
        i_0 = jnp.zeros((_L,), jnp.int32)

        def scan_e(e, carry):
            m1, m2, i1, i2 = carry
            v = lgt_v[e, pl.ds(off, _L)]
            gt1 = v > m1
            gt2 = v > m2
            m2 = jnp.where(gt1, m1, jnp.where(gt2, v, m2))
            i2 = jnp.where(gt1, i1, jnp.where(gt2, e, i2))
            m1 = jnp.where(gt1, v, m1)
            i1 = jnp.where(gt1, e, i1)
            return m1, m2, i1, i2

        m1, m2, i1, i2 = lax.fori_loop(
            1, NUM_EXPERTS, scan_e, (m1_0, m2_0, i_0, i_0))
        t = jnp.exp(m2 - m1)
        denom = t + 1.0
        g1 = 1.0 / denom
        g2 = t / denom
        zero = jnp.zeros((_L,), jnp.float32)

        def write_e(e, _):
            gates_v[e, pl.ds(off, _L)] = jnp.where(
                i1 == e, g1, jnp.where(i2 == e, g2, zero))
            return 0

        lax.fori_loop(0, NUM_EXPERTS, write_e, 0)
        idx_v[0, pl.ds(off, _L)] = i1
        idx_v[1, pl.ds(off, _L)] = i2
        return 0

    lax.fori_loop(0, _RPW // _L, chunk, 0)
    pltpu.sync_copy(gates_v, gates_hbm.at[:, pl.ds(base, _RPW)])
    pltpu.sync_copy(idx_v, idx_hbm.at[:, pl.ds(base, _RPW)])


def kernel(x, W1, b1, W2, b2, W3, b3):
    logits_t = _mlp_logits_t(x, W1, b1, W2, b2, W3, b3)
    gates_t, idx_t = _route_sc(logits_t)
    return gates_t.T, idx_t.T


# speedup vs baseline: 1.0308x; 1.0308x over previous
import functools

import jax
import jax.numpy as jnp
from jax import lax
from jax.experimental import pallas as pl
from jax.experimental.pallas import tpu as pltpu
from jax.experimental.pallas import tpu_sc as plsc

N_TOKENS = 8192
D_MODEL = 768
HIDDEN = 256
NUM_EXPERTS = 16
TOP_K = 2

BLK = 2048

_NC = 2
_NS = 16
_NW = _NC * _NS
_RPW = N_TOKENS // _NW
_L = 16


def _mlp_body(x_ref, w1_ref, b1_ref, w2_ref, b2_ref, w3_ref, b3_ref,
              logits_ref):
    h = jnp.dot(x_ref[...], w1_ref[...], preferred_element_type=jnp.float32)
    h = jnp.maximum(h + b1_ref[...], 0.0)
    h = jnp.dot(h, w2_ref[...], preferred_element_type=jnp.float32)
    h = jnp.maximum(h + b2_ref[...], 0.0)
    lt = lax.dot_general(w3_ref[...], h, (((1,), (1,)), ((), ())),
                         preferred_element_type=jnp.float32)
    logits_ref[...] = lt + jnp.transpose(b3_ref[...])


def _mlp_logits_t(x, W1, b1, W2, b2, W3, b3):
    n = x.shape[0]
    full = lambda i: (0, 0)
    return pl.pallas_call(
        _mlp_body,
        grid=(n // BLK,),
        in_specs=[
            pl.BlockSpec((BLK, D_MODEL), lambda i: (i, 0)),
            pl.BlockSpec((D_MODEL, HIDDEN), full),
            pl.BlockSpec((1, HIDDEN), full),
            pl.BlockSpec((HIDDEN, HIDDEN // 2), full),
            pl.BlockSpec((1, HIDDEN // 2), full),
            pl.BlockSpec((NUM_EXPERTS, HIDDEN // 2), full),
            pl.BlockSpec((1, NUM_EXPERTS), full),
        ],
        out_specs=pl.BlockSpec((NUM_EXPERTS, BLK), lambda i: (0, i)),
        out_shape=jax.ShapeDtypeStruct((NUM_EXPERTS, n), jnp.float32),
    )(x, W1, b1.reshape(1, -1), W2, b2.reshape(1, -1), W3.T,
      b3.reshape(1, -1))


@functools.partial(
    pl.kernel,
    out_type=[
        jax.ShapeDtypeStruct((NUM_EXPERTS, N_TOKENS), jnp.float32),
        jax.ShapeDtypeStruct((TOP_K, N_TOKENS), jnp.int32),
    ],
    mesh=plsc.VectorSubcoreMesh(core_axis_name="c", subcore_axis_name="s"),
    compiler_params=pltpu.CompilerParams(needs_layout_passes=False),
    scratch_types=[
        pltpu.VMEM((NUM_EXPERTS, _RPW), jnp.float32),
        pltpu.VMEM((NUM_EXPERTS, _RPW), jnp.float32),
        pltpu.VMEM((TOP_K, _RPW), jnp.int32),
    ],
)
def _route_sc(lgt_hbm, gates_hbm, idx_hbm, lgt_v, gates_v, idx_v):
    wid = lax.axis_index("s") * _NC + lax.axis_index("c")
    base = wid * _RPW
    pltpu.sync_copy(lgt_hbm.at[:, pl.ds(base, _RPW)], lgt_v)

    def chunk(c, _):
        off = c * _L
        m1 = lgt_v[0, pl.ds(off, _L)]
        m2 = jnp.full((_L,), -jnp.inf, jnp.float32)
        i1 = jnp.zeros((_L,), jnp.int32)
        i2 = jnp.zeros((_L,), jnp.int32)
        for e in range(1, NUM_EXPERTS):
            v = lgt_v[e, pl.ds(off, _L)]
            gt1 = v > m1
            gt2 = v > m2
            m2 = jnp.where(gt1, m1, jnp.where(gt2, v, m2))
            i2 = jnp.where(gt1, i1, jnp.where(gt2, e, i2))
            m1 = jnp.where(gt1, v, m1)
            i1 = jnp.where(gt1, e, i1)
        t = jnp.exp(m2 - m1)
        denom = t + 1.0
        g1 = 1.0 / denom
        g2 = t / denom
        zero = jnp.zeros((_L,), jnp.float32)
        for e in range(NUM_EXPERTS):
            gates_v[e, pl.ds(off, _L)] = jnp.where(
                i1 == e, g1, jnp.where(i2 == e, g2, zero))
        idx_v[0, pl.ds(off, _L)] = i1
        idx_v[1, pl.ds(off, _L)] = i2
        return 0

    lax.fori_loop(0, _RPW // _L, chunk, 0)
    pltpu.sync_copy(gates_v, gates_hbm.at[:, pl.ds(base, _RPW)])
    pltpu.sync_copy(idx_v, idx_hbm.at[:, pl.ds(base, _RPW)])


def kernel(x, W1, b1, W2, b2, W3, b3):
    logits_t = _mlp_logits_t(x, W1, b1, W2, b2, W3, b3)
    gates_t, idx_t = _route_sc(logits_t)
    return gates_t.T, idx_t.T
